# baseline (speedup 1.0000x reference)
import functools

import jax
import jax.numpy as jnp
from jax import lax
from jax.experimental import pallas as pl
from jax.experimental.pallas import tpu as pltpu
from jax.experimental.pallas import tpu_sc as plsc

_NUM_CORES = 2
_NUM_SUBCORES = 16
_NUM_WORKERS = _NUM_CORES * _NUM_SUBCORES

_CHUNK = 128
_NBUF = 7
_LEAD = 4


@functools.partial(jax.jit, static_argnames=("seq", "bt_n", "embed_dim"))
def _sc_gather(idxT2, table, *, seq, bt_n, embed_dim):
    n_blocks = idxT2.shape[0]
    blocks_per_w = n_blocks // _NUM_WORKERS

    mesh = plsc.VectorSubcoreMesh(core_axis_name="c", subcore_axis_name="s")

    @functools.partial(
        pl.kernel,
        out_type=jax.ShapeDtypeStruct((seq, bt_n, _CHUNK, 2 * embed_dim),
                                      jnp.float32),
        mesh=mesh,
        scratch_types=[
            pltpu.VMEM((blocks_per_w, _CHUNK), jnp.int32),
            pltpu.VMEM((_NBUF, _CHUNK, embed_dim), jnp.float32),
            pltpu.SemaphoreType.DMA((_NBUF,)),
            pltpu.SemaphoreType.DMA((_NBUF,)),
        ],
        compiler_params=pltpu.CompilerParams(
            use_tc_tiling_on_sc=False, needs_layout_passes=False),
    )
    def k(idx_hbm, table_hbm, out_hbm, idx_v, rows_v, gsem, wsem):
        wid = lax.axis_index("s") * _NUM_CORES + lax.axis_index("c")
        base = wid * blocks_per_w

        pltpu.sync_copy(idx_hbm.at[pl.ds(base, blocks_per_w)], idx_v)

        def gather(i, b):
            pltpu.async_copy(
                table_hbm.at[idx_v.at[i]], rows_v.at[b], gsem.at[b])

        def gather_wait(i, b):
            pltpu.make_async_copy(
                table_hbm.at[idx_v.at[i]], rows_v.at[b], gsem.at[b]).wait()

        def wb_dst(i):
            j = base + i
            return out_hbm.at[j // bt_n, lax.rem(j, bt_n), :,
                              pl.ds(0, embed_dim)]

        def writeback(i, b):
            pltpu.async_copy(rows_v.at[b], wb_dst(i), wsem.at[b])

        def writeback_wait(i, b):
            pltpu.make_async_copy(rows_v.at[b], wb_dst(i), wsem.at[b]).wait()

        for i in range(_LEAD):
            gather(i, i % _NBUF)

        def body(B, carry):
            nxt = B + _LEAD

            @pl.when(nxt < blocks_per_w)
            def _():
                bn = lax.rem(nxt, _NBUF)

                @pl.when(nxt >= _NBUF)
                def _():
                    writeback_wait(nxt - _NBUF, bn)

                gather(nxt, bn)

            b = lax.rem(B, _NBUF)
            gather_wait(B, b)
            writeback(B, b)
            return carry

        lax.fori_loop(0, blocks_per_w, body, 0)

        for i in range(blocks_per_w - _NBUF, blocks_per_w):
            writeback_wait(i, i % _NBUF)

    return k(idxT2, table)


def kernel(inputs, table):
    batch, seq = inputs.shape
    vocab, embed_dim = table.shape
    bt_n = batch // _CHUNK

    idxT2 = inputs.T.reshape(seq * bt_n, _CHUNK).astype(jnp.int32) * 2

    tp = jnp.concatenate([table, table], axis=1)
    t2 = tp.reshape(2 * vocab, embed_dim)

    out4 = _sc_gather(idxT2, t2, seq=seq, bt_n=bt_n, embed_dim=embed_dim)
    y = out4[..., :embed_dim]
    return y.transpose(1, 2, 0, 3).reshape(batch, seq, embed_dim)


# speedup vs baseline: 1.1875x; 1.1875x over previous
import functools

import jax
import jax.numpy as jnp
from jax import lax
from jax.experimental import pallas as pl
from jax.experimental.pallas import tpu as pltpu
from jax.experimental.pallas import tpu_sc as plsc

_NUM_CORES = 2
_NUM_SUBCORES = 16
_NUM_WORKERS = _NUM_CORES * _NUM_SUBCORES

_CHUNK = 128
_NBUF = 7
_LEAD = 4


@functools.partial(jax.jit, static_argnames=("seq", "bt_n", "embed_dim"))
def _sc_gather(idxT2, table, *, seq, bt_n, embed_dim):
    n_blocks = idxT2.shape[0]
    blocks_per_w = n_blocks // _NUM_WORKERS

    mesh = plsc.VectorSubcoreMesh(core_axis_name="c", subcore_axis_name="s")

    @functools.partial(
        pl.kernel,
        out_type=jax.ShapeDtypeStruct((seq, bt_n, _CHUNK, 2 * embed_dim),
                                      jnp.float32),
        mesh=mesh,
        scratch_types=[
            pltpu.VMEM((blocks_per_w, _CHUNK), jnp.int32),
            pltpu.VMEM((_NBUF, _CHUNK, embed_dim), jnp.float32),
            pltpu.SemaphoreType.DMA((_NBUF,)),
            pltpu.SemaphoreType.DMA((_NBUF,)),
        ],
        compiler_params=pltpu.CompilerParams(
            use_tc_tiling_on_sc=False, needs_layout_passes=False),
    )
    def k(idx_hbm, table_hbm, out_hbm, idx_v, rows_v, gsem, wsem):
        wid = lax.axis_index("s") * _NUM_CORES + lax.axis_index("c")
        base = wid * blocks_per_w

        pltpu.sync_copy(idx_hbm.at[pl.ds(base, blocks_per_w)], idx_v)

        def gather(i, b):
            pltpu.async_copy(
                table_hbm.at[idx_v.at[i]], rows_v.at[b], gsem.at[b])

        def gather_wait(i, b):
            pltpu.make_async_copy(
                table_hbm.at[idx_v.at[i]], rows_v.at[b], gsem.at[b]).wait()

        def wb_dst(i):
            j = base + i
            return out_hbm.at[j // bt_n, lax.rem(j, bt_n), :,
                              pl.ds(0, embed_dim)]

        def writeback(i, b):
            pltpu.async_copy(rows_v.at[b], wb_dst(i), wsem.at[b])

        def writeback_wait(i, b):
            pltpu.make_async_copy(rows_v.at[b], wb_dst(i), wsem.at[b]).wait()

        for i in range(_LEAD):
            gather(i, i % _NBUF)

        def body(B, carry):
            nxt = B + _LEAD

            @pl.when(nxt < blocks_per_w)
            def _():
                bn = lax.rem(nxt, _NBUF)

                @pl.when(nxt >= _NBUF)
                def _():
                    writeback_wait(nxt - _NBUF, bn)

                gather(nxt, bn)

            b = lax.rem(B, _NBUF)
            gather_wait(B, b)
            writeback(B, b)
            return carry

        lax.fori_loop(0, blocks_per_w, body, 0)

        for i in range(blocks_per_w - _NBUF, blocks_per_w):
            writeback_wait(i, i % _NBUF)

    return k(idxT2, table)


def kernel(inputs, table):
    batch, seq = inputs.shape
    vocab, embed_dim = table.shape
    bt_n = batch // _CHUNK

    idxT2 = inputs.T.reshape(seq * bt_n, _CHUNK).astype(jnp.int32) * 2

    tp = jnp.pad(table, ((0, 0), (0, embed_dim)))
    t2 = tp.reshape(2 * vocab, embed_dim)

    out4 = _sc_gather(idxT2, t2, seq=seq, bt_n=bt_n, embed_dim=embed_dim)
    y = out4[..., :embed_dim]
    return y.transpose(1, 2, 0, 3).reshape(batch, seq, embed_dim)
